# baseline (speedup 1.0000x reference)
import functools

import jax
import jax.numpy as jnp
from jax import lax
from jax.experimental import pallas as pl
from jax.experimental.pallas import tpu as pltpu
from jax.experimental.pallas import tpu_sc as plsc

VOCAB = 1000000
DIM = 64
B = 4096
L = 200
N_TOTAL = B * L

NC = 2
NS = 16
NW = NC * NS

PER_W = N_TOTAL // NW
IDX_ROW = 128
KCH = 4
G = KCH * IDX_ROW
N_OUTER = PER_W // G

_mesh = plsc.VectorSubcoreMesh(core_axis_name="c", subcore_axis_name="s")


@functools.partial(
    pl.kernel,
    mesh=_mesh,
    out_type=jax.ShapeDtypeStruct((N_TOTAL, DIM), jnp.float32),
    scratch_types=[
        pltpu.VMEM((KCH, IDX_ROW), jnp.int32),
        pltpu.VMEM((G, DIM), jnp.float32),
        pltpu.SemaphoreType.DMA,
    ],
    compiler_params=pltpu.CompilerParams(use_tc_tiling_on_sc=False),
)
def _emb_lookup(x_hbm, table_hbm, out_hbm, idx_v, rows_v, sem):
    wid = lax.axis_index("s") * NC + lax.axis_index("c")
    base = wid * PER_W

    def chunk(c, carry):
        pltpu.sync_copy(x_hbm.at[wid, c], idx_v)
        cps = [
            pltpu.async_copy(
                table_hbm.at[idx_v.at[j]],
                rows_v.at[pl.ds(j * IDX_ROW, IDX_ROW)],
                sem,
            )
            for j in range(KCH)
        ]
        for cp in cps:
            cp.wait()
        pltpu.sync_copy(rows_v, out_hbm.at[pl.ds(base + c * G, G)])
        return carry

    lax.fori_loop(0, N_OUTER, chunk, 0)


def kernel(x, table):
    x_flat = x.reshape(NW, N_OUTER, KCH, IDX_ROW).astype(jnp.int32)
    out = _emb_lookup(x_flat, table)
    return out.reshape(B, L, DIM)
